# Initial kernel scaffold; baseline (speedup 1.0000x reference)
import jax
import jax.numpy as jnp
from jax.experimental import pallas as pl


def kernel(weight, edge_index_0, edge_value_0, edge_index_1, edge_value_1, edge_index_2, edge_value_2, edge_index_3, edge_value_3, edge_index_4, edge_value_4):
    raise NotImplementedError("write your pallas kernel here")


# speedup vs baseline: 37.5899x; 37.5899x over previous
import functools

import jax
import jax.numpy as jnp
from jax import lax
from jax.experimental import pallas as pl
from jax.experimental.pallas import tpu as pltpu

NUM_NODES = 100000
IN_CHANNELS = 5
OUT_CHANNELS = 2
E = 1600000
NT = IN_CHANNELS * E
B = 1280
G = NT // B
NTPAD = NT + 2 * B
W = B + 256


def _coalesce_body(w_ref, r_ref, c_ref, ev_ref, rel_ref,
                   out_ref,
                   yva, yvb, zv, cv, gprev_s, g_s, cr_s, cc_s, s0_s, s1_s,
                   sem_data, sem_zero):
    i = pl.program_id(0)

    @pl.when(i == 0)
    def _init():
        g_s[0] = 0
        gprev_s[0] = 0
        cr_s[0] = -1
        cc_s[0] = -1
        s0_s[0] = 0.0
        s1_s[0] = 0.0
        zv[...] = jnp.zeros_like(zv)

    r = r_ref[0]
    c = c_ref[0]
    ev = ev_ref[0]
    rel = rel_ref[0]

    w = [[w_ref[ch, j] for j in range(IN_CHANNELS)] for ch in range(OUT_CHANNELS)]
    filt = []
    for ch in range(OUT_CHANNELS):
        m = w[ch][0]
        for j in range(1, IN_CHANNELS):
            m = jnp.maximum(m, w[ch][j])
        e = [jnp.exp(w[ch][j] - m) for j in range(IN_CHANNELS)]
        tot = e[0]
        for j in range(1, IN_CHANNELS):
            tot = tot + e[j]
        filt.append([e[j] / tot for j in range(IN_CHANNELS)])

    sc0 = jnp.full(rel.shape, filt[0][0], jnp.float32)
    sc1 = jnp.full(rel.shape, filt[1][0], jnp.float32)
    for j in range(1, IN_CHANNELS):
        sc0 = jnp.where(rel == j, filt[0][j], sc0)
        sc1 = jnp.where(rel == j, filt[1][j], sc1)
    v0 = ev * sc0
    v1 = ev * sc1

    lane = lax.broadcasted_iota(jnp.int32, (1, B), 1)
    pr = pltpu.roll(r, jnp.int32(1), 1)
    pc = pltpu.roll(c, jnp.int32(1), 1)
    first = lane == 0
    pr = jnp.where(first, cr_s[0], pr)
    pc = jnp.where(first, cc_s[0], pc)
    fli = ((r != pr) | (c != pc)).astype(jnp.int32)
    ids = fli
    s = 1
    while s < B:
        ids = ids + jnp.where(lane >= s, pltpu.roll(ids, jnp.int32(s), 1), 0)
        s *= 2
    flf = fli.astype(jnp.float32)

    rf = r.astype(jnp.float32) * flf
    cf = c.astype(jnp.float32) * flf
    x4 = jnp.concatenate([rf, cf, v0, v1], axis=0)

    kio = lax.broadcasted_iota(jnp.int32, (B + 1, B), 0)
    oht = (kio == jnp.broadcast_to(ids, (B + 1, B))).astype(jnp.float32)

    y = lax.dot_general(x4, oht, (((1,), (1,)), ((), ())),
                        precision=lax.Precision.HIGHEST,
                        preferred_element_type=jnp.float32)

    ci = lax.broadcasted_iota(jnp.int32, (4, 1), 0)
    carry4 = jnp.where(ci == 0, cr_s[0].astype(jnp.float32),
             jnp.where(ci == 1, cc_s[0].astype(jnp.float32),
             jnp.where(ci == 2, s0_s[0], s1_s[0])))
    slot0 = (lax.broadcasted_iota(jnp.int32, (1, B + 1), 1) == 0).astype(jnp.float32)
    y = y + carry4 * slot0

    last_id = jnp.max(ids)
    lastmask = (lane == (B - 1))
    r_last = jnp.max(jnp.where(lastmask, r, -1))
    c_last = jnp.max(jnp.where(lastmask, c, -1))
    kmask = (lax.broadcasted_iota(jnp.int32, (1, B + 1), 1) == last_id).astype(jnp.float32)
    s0_new = jnp.sum(y[2:3, :] * kmask)
    s1_new = jnp.sum(y[3:4, :] * kmask)

    g = g_s[0]

    ga = pl.multiple_of((g // 128) * 128, 128)
    off = g - ga
    lw = lax.broadcasted_iota(jnp.int32, (4, W), 1)
    cache = cv[...]
    yw = pltpu.roll(
        jnp.concatenate([y, jnp.zeros((4, W - (B + 1)), jnp.float32)], axis=1),
        off, 1)
    cw = pltpu.roll(
        jnp.concatenate([cache, jnp.zeros((4, W - 128), jnp.float32)], axis=1),
        off + (W - 128), 1)
    cw = jnp.where(lw < off, cw, 0.0)
    w_win = cw + yw

    comb = jnp.concatenate(
        [cache, y, jnp.zeros((4, W - 128 - (B + 1)), jnp.float32)], axis=1)
    cache_new = pltpu.roll(comb, W - last_id, 1)[:, :128]
    cv[...] = cache_new

    sl = lax.rem(i, jnp.int32(2))

    @pl.when(sl == 0)
    def _sta():
        yva[...] = w_win

    @pl.when(sl == 1)
    def _stb():
        yvb[...] = w_win

    zcp = pltpu.make_async_copy(zv, out_ref.at[:, pl.ds((i + 2) * B, B)], sem_zero)
    zcp.start()

    @pl.when(i == 0)
    def _zfirst():
        zcp2 = pltpu.make_async_copy(zv, out_ref.at[:, pl.ds(B, B)], sem_zero)
        zcp2.start()
        zcp2.wait()

    zcp.wait()

    @pl.when(i > 0)
    def _drain():
        pltpu.make_async_copy(
            yva, out_ref.at[:, pl.ds(pl.multiple_of(gprev_s[0], 128), W)], sem_data).wait()

    @pl.when(sl == 0)
    def _cpa():
        pltpu.make_async_copy(yva, out_ref.at[:, pl.ds(ga, W)], sem_data).start()

    @pl.when(sl == 1)
    def _cpb():
        pltpu.make_async_copy(yvb, out_ref.at[:, pl.ds(ga, W)], sem_data).start()

    @pl.when(i == G - 1)
    def _final_drain():
        pltpu.make_async_copy(
            yva, out_ref.at[:, pl.ds(ga, W)], sem_data).wait()

    gprev_s[0] = ga
    g_s[0] = g + last_id
    cr_s[0] = r_last
    cc_s[0] = c_last
    s0_s[0] = s0_new
    s1_s[0] = s1_new


@functools.partial(jax.jit, static_argnames=())
def _gtconv_impl(weight, rows, cols, evals, rels):
    rs, cs, evs, rls = lax.sort((rows, cols, evals, rels), num_keys=2)

    out = pl.pallas_call(
        _coalesce_body,
        grid=(G,),
        in_specs=[
            pl.BlockSpec((OUT_CHANNELS, IN_CHANNELS), lambda i: (i * 0, i * 0),
                         memory_space=pltpu.SMEM),
            pl.BlockSpec((1, 1, B), lambda i: (i, i * 0, i * 0)),
            pl.BlockSpec((1, 1, B), lambda i: (i, i * 0, i * 0)),
            pl.BlockSpec((1, 1, B), lambda i: (i, i * 0, i * 0)),
            pl.BlockSpec((1, 1, B), lambda i: (i, i * 0, i * 0)),
        ],
        out_specs=pl.BlockSpec(memory_space=pl.ANY),
        out_shape=jax.ShapeDtypeStruct((4, NTPAD), jnp.float32),
        scratch_shapes=[
            pltpu.VMEM((4, W), jnp.float32),
            pltpu.VMEM((4, W), jnp.float32),
            pltpu.VMEM((4, B), jnp.float32),
            pltpu.VMEM((4, 128), jnp.float32),
            pltpu.SMEM((1,), jnp.int32),
            pltpu.SMEM((1,), jnp.int32),
            pltpu.SMEM((1,), jnp.int32),
            pltpu.SMEM((1,), jnp.int32),
            pltpu.SMEM((1,), jnp.float32),
            pltpu.SMEM((1,), jnp.float32),
            pltpu.SemaphoreType.DMA,
            pltpu.SemaphoreType.DMA,
        ],
        compiler_params=pltpu.CompilerParams(
            dimension_semantics=("arbitrary",),
        ),
    )(
        weight,
        rs.reshape(G, 1, B),
        cs.reshape(G, 1, B),
        evs.reshape(G, 1, B),
        rls.reshape(G, 1, B),
    )

    rows_out = out[0, 1:NT + 1].astype(jnp.int64)
    cols_out = out[1, 1:NT + 1].astype(jnp.int64)
    v0 = out[2, 1:NT + 1]
    v1 = out[3, 1:NT + 1]
    index = jnp.stack([rows_out, cols_out])
    return (index, v0), (index, v1)


def kernel(weight, edge_index_0, edge_value_0, edge_index_1, edge_value_1,
           edge_index_2, edge_value_2, edge_index_3, edge_value_3,
           edge_index_4, edge_value_4):
    eis = [edge_index_0, edge_index_1, edge_index_2, edge_index_3, edge_index_4]
    evs = [edge_value_0, edge_value_1, edge_value_2, edge_value_3, edge_value_4]
    rows = jnp.concatenate([ei[0].astype(jnp.int32) for ei in eis])
    cols = jnp.concatenate([ei[1].astype(jnp.int32) for ei in eis])
    vals = jnp.concatenate(evs)
    rels = jnp.concatenate(
        [jnp.full((E,), j, jnp.int32) for j in range(IN_CHANNELS)])
    return _gtconv_impl(weight, rows, cols, vals, rels)
